# baseline (speedup 1.0000x reference)
import functools

import jax
import jax.numpy as jnp
from jax import lax
from jax.experimental import pallas as pl
from jax.experimental.pallas import tpu as pltpu
from jax.experimental.pallas import tpu_sc as plsc

N = 10000
E = 320000
D = 128

NC = 2
NS = 16
NW = NC * NS
EPW = E // NW
SUB = 80
NSUB = 1
BIG = SUB * NSUB
NBIG = EPW // BIG
RPT = 624
TAIL = N - NS * RPT
ZR = 48
NSLOT = 4


def _sc_scatter_body(edge_hbm, recv_hbm, out_hbm, idx_v, rows_v, zbuf, shared,
                     sem_g0, sem_g1, sem_g2, sem_g3,
                     sem_s0, sem_s1, sem_s2, sem_s3):
    c = lax.axis_index("c")
    s = lax.axis_index("s")
    wid = c * NS + s
    sem_g = (sem_g0, sem_g1, sem_g2, sem_g3)
    sem_s = (sem_s0, sem_s1, sem_s2, sem_s3)

    def zero_row(r, carry):
        for q in range(D // 16):
            zbuf[r, 16 * q:16 * (q + 1)] = jnp.zeros((16,), jnp.float32)
        return carry

    lax.fori_loop(0, ZR, zero_row, 0)
    for k in range(RPT // ZR):
        pltpu.sync_copy(zbuf, shared.at[pl.ds(s * RPT + k * ZR, ZR)])

    @pl.when(s == NS - 1)
    def _():
        pltpu.sync_copy(zbuf.at[pl.ds(0, TAIL)], shared.at[pl.ds(NS * RPT, TAIL)])

    plsc.subcore_barrier()

    ebase = wid * EPW

    def gstart(ci, slot):
        b = ebase + ci * BIG
        pltpu.async_copy(edge_hbm.at[pl.ds(b, BIG), :], rows_v.at[slot],
                         sem_g[slot])
        for j in range(NSUB):
            pltpu.async_copy(recv_hbm.at[pl.ds(E + b + j * SUB, SUB)],
                             idx_v.at[slot, j], sem_g[slot])

    def gwait(slot):
        pltpu.make_async_copy(edge_hbm.at[pl.ds(0, BIG), :], rows_v.at[slot],
                              sem_g[slot]).wait()
        for j in range(NSUB):
            pltpu.make_async_copy(recv_hbm.at[pl.ds(0, SUB)],
                                  idx_v.at[slot, j], sem_g[slot]).wait()

    def sstart(slot):
        for j in range(NSUB):
            pltpu.async_copy(rows_v.at[slot, pl.ds(j * SUB, SUB)],
                             shared.at[idx_v.at[slot, j]], sem_s[slot],
                             add=True)

    def swait(slot):
        for j in range(NSUB):
            pltpu.make_async_copy(rows_v.at[slot, pl.ds(j * SUB, SUB)],
                                  shared.at[idx_v.at[slot, j]],
                                  sem_s[slot]).wait()

    gstart(0, 0)
    gstart(1, 1)

    gwait(0); sstart(0); gstart(2, 2)
    gwait(1); sstart(1); gstart(3, 3)
    gwait(2); sstart(2); swait(0); gstart(4, 0)
    gwait(3); sstart(3); swait(1); gstart(5, 1)

    def rnd(t, carry):
        for sl in range(NSLOT):
            ci = t * NSLOT + sl
            sl2 = (sl + 2) % NSLOT
            gwait(sl)
            sstart(sl)

            @pl.when(ci + 2 < NBIG)
            def _():
                swait(sl2)
                gstart(ci + 2, sl2)
        return carry

    lax.fori_loop(1, 1 + (NBIG - NSLOT - 1) // NSLOT, rnd, 0)
    gwait(0)
    sstart(0)
    for sl in range(NSLOT):
        swait(sl)
    plsc.subcore_barrier()

    pltpu.sync_copy(shared.at[pl.ds(s * RPT, RPT)],
                    out_hbm.at[c, pl.ds(s * RPT, RPT)])

    @pl.when(s == NS - 1)
    def _():
        pltpu.sync_copy(shared.at[pl.ds(NS * RPT, TAIL)],
                        out_hbm.at[c, pl.ds(NS * RPT, TAIL)])


_sc_scatter = functools.partial(
    pl.kernel,
    mesh=plsc.VectorSubcoreMesh(core_axis_name="c", subcore_axis_name="s"),
    out_type=jax.ShapeDtypeStruct((NC, N, D), jnp.float32),
    scratch_types=[
        pltpu.VMEM((NSLOT, NSUB, SUB), jnp.int32),
        pltpu.VMEM((NSLOT, BIG, D), jnp.float32),
        pltpu.VMEM((ZR, D), jnp.float32),
        pltpu.VMEM_SHARED((N, D), jnp.float32),
    ] + [pltpu.SemaphoreType.DMA] * (2 * NSLOT),
)(_sc_scatter_body)


def _tc_mlp_body(node_ref, p_ref, w1a_ref, w1b_ref, b1_ref, w2_ref, b2_ref,
                 out_ref):
    agg = p_ref[0] + p_ref[1]
    colsum = jnp.sum(agg, axis=0, keepdims=True)
    bias = b1_ref[...] - jnp.dot(colsum * (1.0 / N), w1b_ref[...],
                                 preferred_element_type=jnp.float32)
    h = jnp.dot(node_ref[...], w1a_ref[...],
                preferred_element_type=jnp.float32)
    h += jnp.dot(agg, w1b_ref[...], preferred_element_type=jnp.float32)
    h = jnp.maximum(h + bias, 0.0)
    out_ref[...] = jnp.dot(h, w2_ref[...],
                           preferred_element_type=jnp.float32) + b2_ref[...]


_tc_mlp = pl.pallas_call(
    _tc_mlp_body,
    out_shape=jax.ShapeDtypeStruct((N, D), jnp.float32),
)


def _copy_body(src_ref, dst_ref):
    dst_ref[...] = src_ref[...]


CB = 8000

_copy_edges = pl.pallas_call(
    _copy_body,
    grid=(E // CB,),
    in_specs=[pl.BlockSpec((CB, D), lambda i: (i, 0))],
    out_specs=pl.BlockSpec((CB, D), lambda i: (i, 0)),
    out_shape=jax.ShapeDtypeStruct((E, D), jnp.float32),
)


@jax.jit
def kernel(node_attr, edge_index, edge_attr, W1, b1, W2, b2):
    partials = _sc_scatter(edge_attr, edge_index.reshape(2 * E))
    x = _tc_mlp(node_attr, partials, W1[:D], W1[D:], b1.reshape(1, D),
                W2, b2.reshape(1, D))
    ea = _copy_edges(edge_attr)
    return (x, edge_index, ea)


# speedup vs baseline: 1.0375x; 1.0375x over previous
import functools

import jax
import jax.numpy as jnp
from jax import lax
from jax.experimental import pallas as pl
from jax.experimental.pallas import tpu as pltpu
from jax.experimental.pallas import tpu_sc as plsc

N = 10000
E = 320000
D = 128

NC = 2
NS = 16
NW = NC * NS
EPW = E // NW
SUB = 80
NSUB = 1
BIG = SUB * NSUB
NBIG = EPW // BIG
RPT = 624
TAIL = N - NS * RPT
ZR = 48
NSLOT = 4


def _sc_scatter_body(edge_hbm, recv_hbm, out_hbm, idx_v, rows_v, zbuf, shared,
                     sem_g0, sem_g1, sem_g2, sem_g3,
                     sem_s0, sem_s1, sem_s2, sem_s3):
    c = lax.axis_index("c")
    s = lax.axis_index("s")
    wid = c * NS + s
    sem_g = (sem_g0, sem_g1, sem_g2, sem_g3)
    sem_s = (sem_s0, sem_s1, sem_s2, sem_s3)

    def zero_row(r, carry):
        for q in range(D // 16):
            zbuf[r, 16 * q:16 * (q + 1)] = jnp.zeros((16,), jnp.float32)
        return carry

    lax.fori_loop(0, ZR, zero_row, 0)
    for k in range(RPT // ZR):
        pltpu.sync_copy(zbuf, shared.at[pl.ds(s * RPT + k * ZR, ZR)])

    @pl.when(s == NS - 1)
    def _():
        pltpu.sync_copy(zbuf.at[pl.ds(0, TAIL)], shared.at[pl.ds(NS * RPT, TAIL)])

    plsc.subcore_barrier()

    ebase = wid * EPW

    def gstart(ci, slot):
        b = ebase + ci * BIG
        pltpu.async_copy(edge_hbm.at[pl.ds(b, BIG), :], rows_v.at[slot],
                         sem_g[slot])
        for j in range(NSUB):
            pltpu.async_copy(recv_hbm.at[pl.ds(E + b + j * SUB, SUB)],
                             idx_v.at[slot, j], sem_g[slot])

    def gwait(slot):
        pltpu.make_async_copy(edge_hbm.at[pl.ds(0, BIG), :], rows_v.at[slot],
                              sem_g[slot]).wait()
        for j in range(NSUB):
            pltpu.make_async_copy(recv_hbm.at[pl.ds(0, SUB)],
                                  idx_v.at[slot, j], sem_g[slot]).wait()

    def sstart(slot):
        for j in range(NSUB):
            pltpu.async_copy(rows_v.at[slot, pl.ds(j * SUB, SUB)],
                             shared.at[idx_v.at[slot, j]], sem_s[slot],
                             add=True)

    def swait(slot):
        for j in range(NSUB):
            pltpu.make_async_copy(rows_v.at[slot, pl.ds(j * SUB, SUB)],
                                  shared.at[idx_v.at[slot, j]],
                                  sem_s[slot]).wait()

    gstart(0, 0)
    gstart(1, 1)

    gwait(0); sstart(0); gstart(2, 2)
    gwait(1); sstart(1); gstart(3, 3)
    gwait(2); sstart(2); swait(0); gstart(4, 0)
    gwait(3); sstart(3); swait(1); gstart(5, 1)

    def rnd(t, carry):
        for sl in range(NSLOT):
            ci = t * NSLOT + sl
            sl2 = (sl + 2) % NSLOT
            gwait(sl)
            sstart(sl)

            @pl.when(ci + 2 < NBIG)
            def _():
                swait(sl2)
                gstart(ci + 2, sl2)
        return carry

    lax.fori_loop(1, 1 + (NBIG - NSLOT - 1) // NSLOT, rnd, 0)
    gwait(0)
    sstart(0)
    for sl in range(NSLOT):
        swait(sl)
    plsc.subcore_barrier()

    pltpu.sync_copy(shared.at[pl.ds(s * RPT, RPT)],
                    out_hbm.at[c, pl.ds(s * RPT, RPT)])

    @pl.when(s == NS - 1)
    def _():
        pltpu.sync_copy(shared.at[pl.ds(NS * RPT, TAIL)],
                        out_hbm.at[c, pl.ds(NS * RPT, TAIL)])


_sc_scatter = functools.partial(
    pl.kernel,
    mesh=plsc.VectorSubcoreMesh(core_axis_name="c", subcore_axis_name="s"),
    out_type=jax.ShapeDtypeStruct((NC, N, D), jnp.float32),
    scratch_types=[
        pltpu.VMEM((NSLOT, NSUB, SUB), jnp.int32),
        pltpu.VMEM((NSLOT, BIG, D), jnp.float32),
        pltpu.VMEM((ZR, D), jnp.float32),
        pltpu.VMEM_SHARED((N, D), jnp.float32),
    ] + [pltpu.SemaphoreType.DMA] * (2 * NSLOT),
)(_sc_scatter_body)


def _tc_mlp_body(node_ref, p_ref, w1a_ref, w1b_ref, b1_ref, w2_ref, b2_ref,
                 out_ref):
    agg = p_ref[0] + p_ref[1]
    colsum = jnp.sum(agg, axis=0, keepdims=True)
    bias = b1_ref[...] - jnp.dot(colsum * (1.0 / N), w1b_ref[...],
                                 preferred_element_type=jnp.float32)
    h = jnp.dot(node_ref[...], w1a_ref[...],
                preferred_element_type=jnp.float32)
    h += jnp.dot(agg, w1b_ref[...], preferred_element_type=jnp.float32)
    h = jnp.maximum(h + bias, 0.0)
    out_ref[...] = jnp.dot(h, w2_ref[...],
                           preferred_element_type=jnp.float32) + b2_ref[...]


_tc_mlp = pl.pallas_call(
    _tc_mlp_body,
    out_shape=jax.ShapeDtypeStruct((N, D), jnp.float32),
)


def _copy_body(src_ref, dst_ref):
    dst_ref[...] = src_ref[...]


CB = 3200

_copy_edges = pl.pallas_call(
    _copy_body,
    grid=(E // CB,),
    in_specs=[pl.BlockSpec((CB, D), lambda i: (i, 0))],
    out_specs=pl.BlockSpec((CB, D), lambda i: (i, 0)),
    out_shape=jax.ShapeDtypeStruct((E, D), jnp.float32),
)


@jax.jit
def kernel(node_attr, edge_index, edge_attr, W1, b1, W2, b2):
    partials = _sc_scatter(edge_attr, edge_index.reshape(2 * E))
    x = _tc_mlp(node_attr, partials, W1[:D], W1[D:], b1.reshape(1, D),
                W2, b2.reshape(1, D))
    ea = _copy_edges(edge_attr)
    return (x, edge_index, ea)


# speedup vs baseline: 1.0393x; 1.0018x over previous
import functools

import jax
import jax.numpy as jnp
from jax import lax
from jax.experimental import pallas as pl
from jax.experimental.pallas import tpu as pltpu
from jax.experimental.pallas import tpu_sc as plsc

N = 10000
E = 320000
D = 128

NC = 2
NS = 16
NW = NC * NS
EPW = E // NW
SUB = 80
NSUB = 1
BIG = SUB * NSUB
NBIG = EPW // BIG
RPT = 624
TAIL = N - NS * RPT
ZR = 48
NSLOT = 4


def _sc_scatter_body(edge_hbm, recv_hbm, out_hbm, idx_v, rows_v, zbuf, shared,
                     sem_g0, sem_g1, sem_g2, sem_g3,
                     sem_s0, sem_s1, sem_s2, sem_s3):
    c = lax.axis_index("c")
    s = lax.axis_index("s")
    wid = c * NS + s
    sem_g = (sem_g0, sem_g1, sem_g2, sem_g3)
    sem_s = (sem_s0, sem_s1, sem_s2, sem_s3)

    def zero_row(r, carry):
        for q in range(D // 16):
            zbuf[r, 16 * q:16 * (q + 1)] = jnp.zeros((16,), jnp.float32)
        return carry

    lax.fori_loop(0, ZR, zero_row, 0)
    for k in range(RPT // ZR):
        pltpu.sync_copy(zbuf, shared.at[pl.ds(s * RPT + k * ZR, ZR)])

    @pl.when(s == NS - 1)
    def _():
        pltpu.sync_copy(zbuf.at[pl.ds(0, TAIL)], shared.at[pl.ds(NS * RPT, TAIL)])

    plsc.subcore_barrier()

    ebase = wid * EPW

    def gstart(ci, slot):
        b = ebase + ci * BIG
        pltpu.async_copy(edge_hbm.at[pl.ds(b, BIG), :], rows_v.at[slot],
                         sem_g[slot])
        for j in range(NSUB):
            pltpu.async_copy(recv_hbm.at[pl.ds(E + b + j * SUB, SUB)],
                             idx_v.at[slot, j], sem_g[slot])

    def gwait(slot):
        pltpu.make_async_copy(edge_hbm.at[pl.ds(0, BIG), :], rows_v.at[slot],
                              sem_g[slot]).wait()
        for j in range(NSUB):
            pltpu.make_async_copy(recv_hbm.at[pl.ds(0, SUB)],
                                  idx_v.at[slot, j], sem_g[slot]).wait()

    def sstart(slot):
        for j in range(NSUB):
            pltpu.async_copy(rows_v.at[slot, pl.ds(j * SUB, SUB)],
                             shared.at[idx_v.at[slot, j]], sem_s[slot],
                             add=True)

    def swait(slot):
        for j in range(NSUB):
            pltpu.make_async_copy(rows_v.at[slot, pl.ds(j * SUB, SUB)],
                                  shared.at[idx_v.at[slot, j]],
                                  sem_s[slot]).wait()

    gstart(0, 0)
    gstart(1, 1)

    gwait(0); sstart(0); gstart(2, 2)
    gwait(1); sstart(1); gstart(3, 3)
    gwait(2); sstart(2); swait(0); gstart(4, 0)
    gwait(3); sstart(3); swait(1); gstart(5, 1)

    def rnd(t, carry):
        for sl in range(NSLOT):
            ci = t * NSLOT + sl
            sl2 = (sl + 2) % NSLOT
            gwait(sl)
            sstart(sl)

            @pl.when(ci + 2 < NBIG)
            def _():
                swait(sl2)
                gstart(ci + 2, sl2)
        return carry

    lax.fori_loop(1, 1 + (NBIG - NSLOT - 1) // NSLOT, rnd, 0)
    gwait(0)
    sstart(0)
    for sl in range(NSLOT):
        swait(sl)
    plsc.subcore_barrier()

    pltpu.sync_copy(shared.at[pl.ds(s * RPT, RPT)],
                    out_hbm.at[c, pl.ds(s * RPT, RPT)])

    @pl.when(s == NS - 1)
    def _():
        pltpu.sync_copy(shared.at[pl.ds(NS * RPT, TAIL)],
                        out_hbm.at[c, pl.ds(NS * RPT, TAIL)])


_sc_scatter = functools.partial(
    pl.kernel,
    mesh=plsc.VectorSubcoreMesh(core_axis_name="c", subcore_axis_name="s"),
    out_type=jax.ShapeDtypeStruct((NC, N, D), jnp.float32),
    scratch_types=[
        pltpu.VMEM((NSLOT, NSUB, SUB), jnp.int32),
        pltpu.VMEM((NSLOT, BIG, D), jnp.float32),
        pltpu.VMEM((ZR, D), jnp.float32),
        pltpu.VMEM_SHARED((N, D), jnp.float32),
    ] + [pltpu.SemaphoreType.DMA] * (2 * NSLOT),
)(_sc_scatter_body)


def _tc_mlp_body(node_ref, p_ref, w1a_ref, w1b_ref, b1_ref, w2_ref, b2_ref,
                 out_ref):
    agg = p_ref[0] + p_ref[1]
    colsum = jnp.sum(agg, axis=0, keepdims=True)
    bias = b1_ref[...] - jnp.dot(colsum * (1.0 / N), w1b_ref[...],
                                 preferred_element_type=jnp.float32)
    h = jnp.dot(node_ref[...], w1a_ref[...],
                preferred_element_type=jnp.float32)
    h += jnp.dot(agg, w1b_ref[...], preferred_element_type=jnp.float32)
    h = jnp.maximum(h + bias, 0.0)
    out_ref[...] = jnp.dot(h, w2_ref[...],
                           preferred_element_type=jnp.float32) + b2_ref[...]


_tc_mlp = pl.pallas_call(
    _tc_mlp_body,
    out_shape=jax.ShapeDtypeStruct((N, D), jnp.float32),
)


def _copy_body(src_ref, dst_ref):
    dst_ref[...] = src_ref[...]


CB = 4000

_copy_edges = pl.pallas_call(
    _copy_body,
    grid=(E // CB,),
    in_specs=[pl.BlockSpec((CB, D), lambda i: (i, 0))],
    out_specs=pl.BlockSpec((CB, D), lambda i: (i, 0)),
    out_shape=jax.ShapeDtypeStruct((E, D), jnp.float32),
)


@jax.jit
def kernel(node_attr, edge_index, edge_attr, W1, b1, W2, b2):
    partials = _sc_scatter(edge_attr, edge_index.reshape(2 * E))
    x = _tc_mlp(node_attr, partials, W1[:D], W1[D:], b1.reshape(1, D),
                W2, b2.reshape(1, D))
    ea = _copy_edges(edge_attr)
    return (x, edge_index, ea)


# speedup vs baseline: 1.2537x; 1.2063x over previous
import functools

import jax
import jax.numpy as jnp
from jax import lax
from jax.experimental import pallas as pl
from jax.experimental.pallas import tpu as pltpu
from jax.experimental.pallas import tpu_sc as plsc

N = 10000
E = 320000
D = 128

NC = 2
NS = 16
NW = NC * NS
EPW = E // NW
SUB = 80
NSUB = 1
BIG = SUB * NSUB
NBIG = EPW // BIG
RPT = 624
TAIL = N - NS * RPT
ZR = 48
NSLOT = 4


def _sc_scatter_body(edge_hbm, recv_hbm, out_hbm, ea_hbm, idx_v, rows_v, zbuf,
                     shared,
                     sem_g0, sem_g1, sem_g2, sem_g3,
                     sem_s0, sem_s1, sem_s2, sem_s3,
                     sem_w0, sem_w1, sem_w2, sem_w3):
    c = lax.axis_index("c")
    s = lax.axis_index("s")
    wid = c * NS + s
    sem_g = (sem_g0, sem_g1, sem_g2, sem_g3)
    sem_s = (sem_s0, sem_s1, sem_s2, sem_s3)
    sem_w = (sem_w0, sem_w1, sem_w2, sem_w3)

    def zero_row(r, carry):
        for q in range(D // 16):
            zbuf[r, 16 * q:16 * (q + 1)] = jnp.zeros((16,), jnp.float32)
        return carry

    lax.fori_loop(0, ZR, zero_row, 0)
    for k in range(RPT // ZR):
        pltpu.sync_copy(zbuf, shared.at[pl.ds(s * RPT + k * ZR, ZR)])

    @pl.when(s == NS - 1)
    def _():
        pltpu.sync_copy(zbuf.at[pl.ds(0, TAIL)], shared.at[pl.ds(NS * RPT, TAIL)])

    plsc.subcore_barrier()

    ebase = wid * EPW

    def gstart(ci, slot):
        b = ebase + ci * BIG
        pltpu.async_copy(edge_hbm.at[pl.ds(b, BIG), :], rows_v.at[slot],
                         sem_g[slot])
        for j in range(NSUB):
            pltpu.async_copy(recv_hbm.at[pl.ds(E + b + j * SUB, SUB)],
                             idx_v.at[slot, j], sem_g[slot])

    def gwait(slot):
        pltpu.make_async_copy(edge_hbm.at[pl.ds(0, BIG), :], rows_v.at[slot],
                              sem_g[slot]).wait()
        for j in range(NSUB):
            pltpu.make_async_copy(recv_hbm.at[pl.ds(0, SUB)],
                                  idx_v.at[slot, j], sem_g[slot]).wait()

    def sstart(ci, slot):
        for j in range(NSUB):
            pltpu.async_copy(rows_v.at[slot, pl.ds(j * SUB, SUB)],
                             shared.at[idx_v.at[slot, j]], sem_s[slot],
                             add=True)
        pltpu.async_copy(rows_v.at[slot],
                         ea_hbm.at[pl.ds(ebase + ci * BIG, BIG), :],
                         sem_w[slot])

    def swait(slot):
        for j in range(NSUB):
            pltpu.make_async_copy(rows_v.at[slot, pl.ds(j * SUB, SUB)],
                                  shared.at[idx_v.at[slot, j]],
                                  sem_s[slot]).wait()
        pltpu.make_async_copy(rows_v.at[slot],
                              ea_hbm.at[pl.ds(0, BIG), :],
                              sem_w[slot]).wait()

    gstart(0, 0)
    gstart(1, 1)

    gwait(0); sstart(0, 0); gstart(2, 2)
    gwait(1); sstart(1, 1); gstart(3, 3)
    gwait(2); sstart(2, 2); swait(0); gstart(4, 0)
    gwait(3); sstart(3, 3); swait(1); gstart(5, 1)

    def rnd(t, carry):
        for sl in range(NSLOT):
            ci = t * NSLOT + sl
            sl2 = (sl + 2) % NSLOT
            gwait(sl)
            sstart(ci, sl)

            @pl.when(ci + 2 < NBIG)
            def _():
                swait(sl2)
                gstart(ci + 2, sl2)
        return carry

    lax.fori_loop(1, 1 + (NBIG - NSLOT - 1) // NSLOT, rnd, 0)
    gwait(0)
    sstart(NBIG - 1, 0)
    for sl in range(NSLOT):
        swait(sl)
    plsc.subcore_barrier()

    pltpu.sync_copy(shared.at[pl.ds(s * RPT, RPT)],
                    out_hbm.at[c, pl.ds(s * RPT, RPT)])

    @pl.when(s == NS - 1)
    def _():
        pltpu.sync_copy(shared.at[pl.ds(NS * RPT, TAIL)],
                        out_hbm.at[c, pl.ds(NS * RPT, TAIL)])


_sc_scatter = functools.partial(
    pl.kernel,
    mesh=plsc.VectorSubcoreMesh(core_axis_name="c", subcore_axis_name="s"),
    out_type=(jax.ShapeDtypeStruct((NC, N, D), jnp.float32),
              jax.ShapeDtypeStruct((E, D), jnp.float32)),
    scratch_types=[
        pltpu.VMEM((NSLOT, NSUB, SUB), jnp.int32),
        pltpu.VMEM((NSLOT, BIG, D), jnp.float32),
        pltpu.VMEM((ZR, D), jnp.float32),
        pltpu.VMEM_SHARED((N, D), jnp.float32),
    ] + [pltpu.SemaphoreType.DMA] * (3 * NSLOT),
)(_sc_scatter_body)


def _tc_mlp_body(node_ref, p_ref, w1a_ref, w1b_ref, b1_ref, w2_ref, b2_ref,
                 out_ref):
    agg = p_ref[0] + p_ref[1]
    colsum = jnp.sum(agg, axis=0, keepdims=True)
    bias = b1_ref[...] - jnp.dot(colsum * (1.0 / N), w1b_ref[...],
                                 preferred_element_type=jnp.float32)
    h = jnp.dot(node_ref[...], w1a_ref[...],
                preferred_element_type=jnp.float32)
    h += jnp.dot(agg, w1b_ref[...], preferred_element_type=jnp.float32)
    h = jnp.maximum(h + bias, 0.0)
    out_ref[...] = jnp.dot(h, w2_ref[...],
                           preferred_element_type=jnp.float32) + b2_ref[...]


_tc_mlp = pl.pallas_call(
    _tc_mlp_body,
    out_shape=jax.ShapeDtypeStruct((N, D), jnp.float32),
)


@jax.jit
def kernel(node_attr, edge_index, edge_attr, W1, b1, W2, b2):
    partials, ea = _sc_scatter(edge_attr, edge_index.reshape(2 * E))
    x = _tc_mlp(node_attr, partials, W1[:D], W1[D:], b1.reshape(1, D),
                W2, b2.reshape(1, D))
    return (x, edge_index, ea)


# speedup vs baseline: 1.2560x; 1.0018x over previous
import functools

import jax
import jax.numpy as jnp
from jax import lax
from jax.experimental import pallas as pl
from jax.experimental.pallas import tpu as pltpu
from jax.experimental.pallas import tpu_sc as plsc

N = 10000
E = 320000
D = 128

NC = 2
NS = 16
NW = NC * NS
EPW = E // NW
SUB = 80
NSUB = 1
BIG = SUB * NSUB
NBIG = EPW // BIG
RPT = 624
TAIL = N - NS * RPT
ZR = 48
NSLOT = 4


def _sc_scatter_body(edge_hbm, recv_hbm, out_hbm, ea_hbm, idx_v, rows_v, zbuf,
                     shared,
                     sem_g0, sem_g1, sem_g2, sem_g3,
                     sem_s0, sem_s1, sem_s2, sem_s3,
                     sem_w0, sem_w1, sem_w2, sem_w3):
    c = lax.axis_index("c")
    s = lax.axis_index("s")
    wid = c * NS + s
    sem_g = (sem_g0, sem_g1, sem_g2, sem_g3)
    sem_s = (sem_s0, sem_s1, sem_s2, sem_s3)
    sem_w = (sem_w0, sem_w1, sem_w2, sem_w3)

    def zero_row(r, carry):
        for q in range(D // 16):
            zbuf[r, 16 * q:16 * (q + 1)] = jnp.zeros((16,), jnp.float32)
        return carry

    lax.fori_loop(0, ZR, zero_row, 0)
    for k in range(RPT // ZR):
        pltpu.sync_copy(zbuf, shared.at[pl.ds(s * RPT + k * ZR, ZR)])

    @pl.when(s == NS - 1)
    def _():
        pltpu.sync_copy(zbuf.at[pl.ds(0, TAIL)], shared.at[pl.ds(NS * RPT, TAIL)])

    plsc.subcore_barrier()

    ebase = wid * EPW

    def gstart(ci, slot):
        b = ebase + ci * BIG
        pltpu.async_copy(edge_hbm.at[pl.ds(b, BIG), :], rows_v.at[slot],
                         sem_g[slot])
        for j in range(NSUB):
            pltpu.async_copy(recv_hbm.at[pl.ds(E + b + j * SUB, SUB)],
                             idx_v.at[slot, j], sem_g[slot])

    def gwait(slot):
        pltpu.make_async_copy(edge_hbm.at[pl.ds(0, BIG), :], rows_v.at[slot],
                              sem_g[slot]).wait()
        for j in range(NSUB):
            pltpu.make_async_copy(recv_hbm.at[pl.ds(0, SUB)],
                                  idx_v.at[slot, j], sem_g[slot]).wait()

    def sstart(ci, slot):
        for j in range(NSUB):
            pltpu.async_copy(rows_v.at[slot, pl.ds(j * SUB, SUB)],
                             shared.at[idx_v.at[slot, j]], sem_s[slot],
                             add=True)
        pltpu.async_copy(rows_v.at[slot],
                         ea_hbm.at[pl.ds(ebase + ci * BIG, BIG), :],
                         sem_w[slot])

    def swait(slot):
        for j in range(NSUB):
            pltpu.make_async_copy(rows_v.at[slot, pl.ds(j * SUB, SUB)],
                                  shared.at[idx_v.at[slot, j]],
                                  sem_s[slot]).wait()
        pltpu.make_async_copy(rows_v.at[slot],
                              ea_hbm.at[pl.ds(0, BIG), :],
                              sem_w[slot]).wait()

    gstart(0, 0)
    gstart(1, 1)

    gwait(0); sstart(0, 0); gstart(2, 2)
    gwait(1); sstart(1, 1); gstart(3, 3)
    gwait(2); sstart(2, 2); swait(0); gstart(4, 0)
    gwait(3); sstart(3, 3); swait(1); gstart(5, 1)

    def rnd(t, carry):
        for sl in range(NSLOT):
            ci = t * NSLOT + sl
            sl2 = (sl + 2) % NSLOT
            gwait(sl)
            sstart(ci, sl)

            @pl.when(ci + 2 < NBIG)
            def _():
                swait(sl2)
                gstart(ci + 2, sl2)
        return carry

    lax.fori_loop(1, 1 + (NBIG - NSLOT - 1) // NSLOT, rnd, 0)
    gwait(0)
    sstart(NBIG - 1, 0)
    for sl in range(NSLOT):
        swait(sl)
    plsc.subcore_barrier()

    pltpu.sync_copy(shared.at[pl.ds(s * RPT, RPT)],
                    out_hbm.at[c, pl.ds(s * RPT, RPT)])

    @pl.when(s == NS - 1)
    def _():
        pltpu.sync_copy(shared.at[pl.ds(NS * RPT, TAIL)],
                        out_hbm.at[c, pl.ds(NS * RPT, TAIL)])


_sc_scatter = functools.partial(
    pl.kernel,
    mesh=plsc.VectorSubcoreMesh(core_axis_name="c", subcore_axis_name="s"),
    out_type=(jax.ShapeDtypeStruct((NC, N, D), jnp.float32),
              jax.ShapeDtypeStruct((E, D), jnp.float32)),
    scratch_types=[
        pltpu.VMEM((NSLOT, NSUB, SUB), jnp.int32),
        pltpu.VMEM((NSLOT, BIG, D), jnp.float32),
        pltpu.VMEM((ZR, D), jnp.float32),
        pltpu.VMEM_SHARED((N, D), jnp.float32),
    ] + [pltpu.SemaphoreType.DMA] * (3 * NSLOT),
)(_sc_scatter_body)


def _tc_pre_body(node_ref, w1a_ref, b1_ref, ha_ref):
    ha_ref[...] = jnp.dot(node_ref[...], w1a_ref[...],
                          preferred_element_type=jnp.float32) + b1_ref[...]


_tc_pre = pl.pallas_call(
    _tc_pre_body,
    out_shape=jax.ShapeDtypeStruct((N, D), jnp.float32),
)


def _tc_mlp_body(ha_ref, p_ref, w1b_ref, w2_ref, b2_ref, out_ref):
    agg = p_ref[0] + p_ref[1]
    colsum = jnp.sum(agg, axis=0, keepdims=True)
    bias = -jnp.dot(colsum * (1.0 / N), w1b_ref[...],
                    preferred_element_type=jnp.float32)
    h = ha_ref[...] + bias
    h += jnp.dot(agg, w1b_ref[...], preferred_element_type=jnp.float32)
    h = jnp.maximum(h, 0.0)
    out_ref[...] = jnp.dot(h, w2_ref[...],
                           preferred_element_type=jnp.float32) + b2_ref[...]


_tc_mlp = pl.pallas_call(
    _tc_mlp_body,
    out_shape=jax.ShapeDtypeStruct((N, D), jnp.float32),
)


@jax.jit
def kernel(node_attr, edge_index, edge_attr, W1, b1, W2, b2):
    partials, ea = _sc_scatter(edge_attr, edge_index.reshape(2 * E))
    ha = _tc_pre(node_attr, W1[:D], b1.reshape(1, D))
    x = _tc_mlp(ha, partials, W1[D:], W2, b2.reshape(1, D))
    return (x, edge_index, ea)
